# Initial kernel scaffold; baseline (speedup 1.0000x reference)
import functools

import jax
import jax.numpy as jnp
from jax import lax
from jax.experimental import pallas as pl
from jax.experimental.pallas import tpu as pltpu
from jax.experimental.pallas import tpu_sc as plsc

_N = 10000
_E = 320000
_D = 128
_H = 128

_NC = 2
_NS = 16
_NW = _NC * _NS
_CB = 128
_K = 80
_EPT = _K * _CB
_EP = _NW * _EPT
_NPAD = 10240
_RPT = _NPAD // _NS

_mesh = plsc.VectorSubcoreMesh(core_axis_name="c", subcore_axis_name="s")


@functools.partial(
    pl.kernel,
    out_type=(
        jax.ShapeDtypeStruct((_NC, _NPAD, 16), jnp.float32),
        jax.ShapeDtypeStruct((_NC, _NPAD, 16), jnp.float32),
    ),
    mesh=_mesh,
    scratch_types=[
        pltpu.VMEM((_K, _CB), jnp.int32),
        pltpu.VMEM((_K, _CB), jnp.int32),
        pltpu.VMEM((_CB, 16), jnp.float32),
        pltpu.VMEM_SHARED((_NPAD, 16), jnp.float32),
        pltpu.VMEM_SHARED((_NPAD, 16), jnp.float32),
    ],
)
def _sc_degrees(src_hbm, dst_hbm, z16_hbm, ones_hbm,
                hs_out, hd_out,
                idx_s, idx_d, ones_v, hist_s, hist_d):
    c = lax.axis_index("c")
    s = lax.axis_index("s")
    wid = c * _NS + s
    rows = pl.ds(s * _RPT, _RPT)
    pltpu.sync_copy(src_hbm.at[wid], idx_s)
    pltpu.sync_copy(dst_hbm.at[wid], idx_d)
    pltpu.sync_copy(ones_hbm, ones_v)
    pltpu.sync_copy(z16_hbm.at[rows], hist_s.at[rows])
    pltpu.sync_copy(z16_hbm.at[rows], hist_d.at[rows])
    plsc.subcore_barrier()

    @pl.loop(0, _K)
    def _(j):
        pltpu.sync_copy(ones_v, hist_s.at[idx_s.at[j]], add=True)
        pltpu.sync_copy(ones_v, hist_d.at[idx_d.at[j]], add=True)

    plsc.subcore_barrier()
    pltpu.sync_copy(hist_s.at[rows], hs_out.at[c, rows])
    pltpu.sync_copy(hist_d.at[rows], hd_out.at[c, rows])


@functools.partial(
    pl.kernel,
    out_type=jax.ShapeDtypeStruct((_NC, _NPAD, _D), jnp.float32),
    mesh=_mesh,
    scratch_types=[
        pltpu.VMEM((_K, _CB), jnp.int32),
        pltpu.VMEM((_K, _CB), jnp.int32),
        pltpu.VMEM((_CB, _D), jnp.float32),
        pltpu.VMEM_SHARED((_NPAD, _D), jnp.float32),
        pltpu.SemaphoreType.DMA,
    ],
)
def _sc_propagate(x_hbm, src_hbm, dst_hbm, z128_hbm,
                  out_hbm,
                  idx_s, idx_d, rowbuf, acc, gsem):
    c = lax.axis_index("c")
    s = lax.axis_index("s")
    wid = c * _NS + s
    rows = pl.ds(s * _RPT, _RPT)
    pltpu.sync_copy(src_hbm.at[wid], idx_s)
    pltpu.sync_copy(dst_hbm.at[wid], idx_d)
    pltpu.sync_copy(z128_hbm.at[rows], acc.at[rows])
    plsc.subcore_barrier()

    @pl.loop(0, _K)
    def _(j):
        pltpu.async_copy(x_hbm.at[idx_s.at[j]], rowbuf, gsem).wait()
        pltpu.sync_copy(rowbuf, acc.at[idx_d.at[j]], add=True)

    plsc.subcore_barrier()
    pltpu.sync_copy(acc.at[rows], out_hbm.at[c, rows])


_BK = 1024


def _norm_from_hist(h_blk):
    deg = h_blk[0, :, 0] + h_blk[1, :, 0]
    return lax.rsqrt(jnp.maximum(deg, 1.0))


def _scale_body(x_ref, hs_ref, o_ref):
    ns = _norm_from_hist(hs_ref[...])
    o_ref[...] = x_ref[...] * ns[:, None]


_tc_scale = pl.pallas_call(
    _scale_body,
    grid=(_NPAD // _BK,),
    in_specs=[
        pl.BlockSpec((_BK, _D), lambda i: (i, 0)),
        pl.BlockSpec((_NC, _BK, 16), lambda i: (0, i, 0)),
    ],
    out_specs=pl.BlockSpec((_BK, _D), lambda i: (i, 0)),
    out_shape=jax.ShapeDtypeStruct((_NPAD, _D), jnp.float32),
)


def _mlp_body(p_ref, hs_ref, hd_ref, w1_ref, w2_ref, o_ref):
    nd = _norm_from_hist(hd_ref[...])
    ns = _norm_from_hist(hs_ref[...])
    agg = (p_ref[0] + p_ref[1]) * nd[:, None]
    h1 = jnp.maximum(
        jnp.dot(agg, w1_ref[...], preferred_element_type=jnp.float32), 0.0)
    o_ref[...] = jnp.dot(h1 * ns[:, None], w2_ref[...],
                         preferred_element_type=jnp.float32)


_tc_mlp = pl.pallas_call(
    _mlp_body,
    grid=(_NPAD // _BK,),
    in_specs=[
        pl.BlockSpec((_NC, _BK, _D), lambda i: (0, i, 0)),
        pl.BlockSpec((_NC, _BK, 16), lambda i: (0, i, 0)),
        pl.BlockSpec((_NC, _BK, 16), lambda i: (0, i, 0)),
        pl.BlockSpec((_D, 2 * _H), lambda i: (0, 0)),
        pl.BlockSpec((2 * _H, _H), lambda i: (0, 0)),
    ],
    out_specs=pl.BlockSpec((_BK, _H), lambda i: (i, 0)),
    out_shape=jax.ShapeDtypeStruct((_NPAD, _H), jnp.float32),
)


def _final_body(p_ref, hd_ref, o_ref):
    nd = _norm_from_hist(hd_ref[...])
    o_ref[...] = (p_ref[0] + p_ref[1]) * nd[:, None]


_tc_final = pl.pallas_call(
    _final_body,
    grid=(_NPAD // _BK,),
    in_specs=[
        pl.BlockSpec((_NC, _BK, _H), lambda i: (0, i, 0)),
        pl.BlockSpec((_NC, _BK, 16), lambda i: (0, i, 0)),
    ],
    out_specs=pl.BlockSpec((_BK, _H), lambda i: (i, 0)),
    out_shape=jax.ShapeDtypeStruct((_NPAD, _H), jnp.float32),
)


def kernel(x, edge_index, W1, W2):
    src = edge_index[0].astype(jnp.int32)
    dst = edge_index[1].astype(jnp.int32)
    pad = jnp.full((_EP - _E,), _N, jnp.int32)
    srcp = jnp.concatenate([src, pad]).reshape(_NW, _K, _CB)
    dstp = jnp.concatenate([dst, pad]).reshape(_NW, _K, _CB)
    xpad = jnp.concatenate(
        [x.astype(jnp.float32), jnp.zeros((_NPAD - _N, _D), jnp.float32)])
    z16 = jnp.zeros((_NPAD, 16), jnp.float32)
    z128 = jnp.zeros((_NPAD, _D), jnp.float32)
    ones16 = jnp.ones((_CB, 16), jnp.float32)

    hs, hd = _sc_degrees(srcp, dstp, z16, ones16)
    xs = _tc_scale(xpad, hs)
    p1 = _sc_propagate(xs, srcp, dstp, z128)
    g = _tc_mlp(p1, hs, hd, W1, W2)
    p2 = _sc_propagate(g, srcp, dstp, z128)
    out = _tc_final(p2, hd)
    return out[:_N]


# speedup vs baseline: 4.4037x; 4.4037x over previous
import functools

import jax
import jax.numpy as jnp
from jax import lax
from jax.experimental import pallas as pl
from jax.experimental.pallas import tpu as pltpu
from jax.experimental.pallas import tpu_sc as plsc

_N = 10000
_E = 320000
_D = 128
_H = 128

_NC = 2
_NS = 16
_NW = _NC * _NS
_CB = 64
_K = 160
_EPT = _K * _CB
_EP = _NW * _EPT
_NPAD = 10240
_RPT = _NPAD // _NS
_HR = _NPAD // _D

_mesh = plsc.VectorSubcoreMesh(core_axis_name="c", subcore_axis_name="s")


_HS = 256


@functools.partial(
    pl.kernel,
    out_type=jax.ShapeDtypeStruct((_NC, _HS, _D), jnp.float32),
    mesh=_mesh,
    scratch_types=[
        pltpu.VMEM((1, _EPT), jnp.int32),
        pltpu.VMEM((1, _EPT), jnp.int32),
        pltpu.VMEM((_HR, _D), jnp.float32),
        pltpu.VMEM((_HR, _D), jnp.float32),
        pltpu.VMEM((2, _HR), jnp.int32),
        pltpu.VMEM_SHARED((_HS, _D), jnp.float32),
    ],
    compiler_params=pltpu.CompilerParams(needs_layout_passes=False),
)
def _sc_degrees(srcf_hbm, dstf_hbm, z128_hbm, ident_hbm,
                h_out,
                idx_s, idx_d, hist_s, hist_d, ident_v, hsh):
    c = lax.axis_index("c")
    s = lax.axis_index("s")
    wid = c * _NS + s
    rows = pl.ds(s * (_HS // _NS), _HS // _NS)
    pltpu.sync_copy(srcf_hbm.at[wid], idx_s)
    pltpu.sync_copy(dstf_hbm.at[wid], idx_d)
    pltpu.sync_copy(ident_hbm, ident_v)
    pltpu.sync_copy(z128_hbm.at[pl.ds(0, _HR)], hist_s)
    pltpu.sync_copy(z128_hbm.at[pl.ds(0, _HR)], hist_d)
    pltpu.sync_copy(z128_hbm.at[rows], hsh.at[rows])

    ones = jnp.full((16,), 1.0, jnp.float32)

    @pl.loop(0, _EPT // 16)
    def _(i):
        sv = idx_s[0, pl.ds(i * 16, 16)]
        dv = idx_d[0, pl.ds(i * 16, 16)]
        plsc.addupdate_scatter(
            hist_s,
            [lax.shift_right_logical(sv, 7), lax.bitwise_and(sv, 127)],
            ones)
        plsc.addupdate_scatter(
            hist_d,
            [lax.shift_right_logical(dv, 7), lax.bitwise_and(dv, 127)],
            ones)

    plsc.subcore_barrier()
    pltpu.sync_copy(hist_s, hsh.at[ident_v.at[0]], add=True)
    pltpu.sync_copy(hist_d, hsh.at[ident_v.at[1]], add=True)
    plsc.subcore_barrier()
    pltpu.sync_copy(hsh.at[rows], h_out.at[c, rows])


_NBUF = 4
_NQ = 8
_QC = _K // _NQ


@functools.partial(
    pl.kernel,
    out_type=jax.ShapeDtypeStruct((_NC, _NPAD, _D), jnp.float32),
    mesh=_mesh,
    scratch_types=[
        pltpu.VMEM((2, _QC, _CB), jnp.int32),
        pltpu.VMEM((2, _QC, _CB), jnp.int32),
        pltpu.VMEM((_CB, _D), jnp.float32),
        pltpu.VMEM((_CB, _D), jnp.float32),
        pltpu.VMEM((_CB, _D), jnp.float32),
        pltpu.VMEM((_CB, _D), jnp.float32),
        pltpu.VMEM_SHARED((_NPAD, _D), jnp.float32),
        pltpu.SemaphoreType.DMA,
        pltpu.SemaphoreType.DMA,
        pltpu.SemaphoreType.DMA,
        pltpu.SemaphoreType.DMA,
        pltpu.SemaphoreType.DMA,
        pltpu.SemaphoreType.DMA,
    ],
)
def _sc_propagate(x_hbm, e_hbm, z128_hbm,
                  out_hbm,
                  eq0, eq1, rbuf0, rbuf1, rbuf2, rbuf3, acc,
                  sem0, sem1, sem2, sem3, isem0, isem1):
    c = lax.axis_index("c")
    s = lax.axis_index("s")
    wid = c * _NS + s
    rows = pl.ds(s * _RPT, _RPT)
    sems = (sem0, sem1, sem2, sem3)
    bufs = (rbuf0, rbuf1, rbuf2, rbuf3)
    eqs = (eq0, eq1)
    isems = (isem0, isem1)
    pltpu.async_copy(e_hbm.at[wid, 0], eq0, isem0)
    pltpu.async_copy(e_hbm.at[wid, 1], eq1, isem1)
    pltpu.sync_copy(z128_hbm.at[rows], acc.at[rows])
    plsc.subcore_barrier()

    for q in range(_NQ):
        eq = eqs[q % 2]
        isem = isems[q % 2]
        pltpu.make_async_copy(
            e_hbm.at[wid, 0], eq, isem).wait()
        for b in range(_NBUF):
            pltpu.async_copy(x_hbm.at[eq.at[0, b]], bufs[b], sems[b])

        @pl.loop(0, _QC, step=_NBUF)
        def _(jj):
            for b in range(_NBUF):
                j = jj + b
                pltpu.make_async_copy(
                    x_hbm.at[pl.ds(0, _CB)], bufs[b], sems[b]).wait()
                pltpu.sync_copy(bufs[b], acc.at[eq.at[1, j]], add=True)
                nxt = j + _NBUF

                @pl.when(nxt < _QC)
                def _():
                    pltpu.async_copy(x_hbm.at[eq.at[0, nxt]], bufs[b],
                                     sems[b])

        if q + 2 < _NQ:
            pltpu.async_copy(e_hbm.at[wid, q + 2], eq, isem)

    plsc.subcore_barrier()
    pltpu.sync_copy(acc.at[rows], out_hbm.at[c, rows])


_BK = 1024
_HB = _BK // _D


def _norm_from_hist(h_blk):
    deg = (h_blk[0] + h_blk[1]).reshape(_BK)
    return lax.rsqrt(jnp.maximum(deg, 1.0))


def _scale_body(x_ref, hs_ref, o_ref):
    ns = _norm_from_hist(hs_ref[...])
    o_ref[...] = x_ref[...] * ns[:, None]


_tc_scale = pl.pallas_call(
    _scale_body,
    grid=(_NPAD // _BK,),
    in_specs=[
        pl.BlockSpec((_BK, _D), lambda i: (i, 0)),
        pl.BlockSpec((_NC, _HB, _D), lambda i: (0, i, 0)),
    ],
    out_specs=pl.BlockSpec((_BK, _D), lambda i: (i, 0)),
    out_shape=jax.ShapeDtypeStruct((_NPAD, _D), jnp.float32),
)


def _mlp_body(p_ref, hs_ref, hd_ref, w1_ref, w2_ref, o_ref):
    nd = _norm_from_hist(hd_ref[...])
    ns = _norm_from_hist(hs_ref[...])
    agg = (p_ref[0] + p_ref[1]) * nd[:, None]
    h1 = jnp.maximum(
        jnp.dot(agg, w1_ref[...], preferred_element_type=jnp.float32), 0.0)
    o_ref[...] = jnp.dot(h1 * ns[:, None], w2_ref[...],
                         preferred_element_type=jnp.float32)


_tc_mlp = pl.pallas_call(
    _mlp_body,
    grid=(_NPAD // _BK,),
    in_specs=[
        pl.BlockSpec((_NC, _BK, _D), lambda i: (0, i, 0)),
        pl.BlockSpec((_NC, _HB, _D), lambda i: (0, i, 0)),
        pl.BlockSpec((_NC, _HB, _D), lambda i: (0, i + 128 // _HB, 0)),
        pl.BlockSpec((_D, 2 * _H), lambda i: (0, 0)),
        pl.BlockSpec((2 * _H, _H), lambda i: (0, 0)),
    ],
    out_specs=pl.BlockSpec((_BK, _H), lambda i: (i, 0)),
    out_shape=jax.ShapeDtypeStruct((_NPAD, _H), jnp.float32),
)


def _final_body(p_ref, hd_ref, o_ref):
    nd = _norm_from_hist(hd_ref[...])
    o_ref[...] = (p_ref[0] + p_ref[1]) * nd[:, None]


_tc_final = pl.pallas_call(
    _final_body,
    grid=(_NPAD // _BK,),
    in_specs=[
        pl.BlockSpec((_NC, _BK, _H), lambda i: (0, i, 0)),
        pl.BlockSpec((_NC, _HB, _D), lambda i: (0, i + 128 // _HB, 0)),
    ],
    out_specs=pl.BlockSpec((_BK, _H), lambda i: (i, 0)),
    out_shape=jax.ShapeDtypeStruct((_NPAD, _H), jnp.float32),
)


def kernel(x, edge_index, W1, W2):
    src = edge_index[0].astype(jnp.int32)
    dst = edge_index[1].astype(jnp.int32)
    pad = jnp.full((_EP - _E,), _N, jnp.int32)
    srcp = jnp.concatenate([src, pad]).reshape(_NW, _K, _CB)
    dstp = jnp.concatenate([dst, pad]).reshape(_NW, _K, _CB)
    srcf = srcp.reshape(_NW, 1, _EPT)
    dstf = dstp.reshape(_NW, 1, _EPT)
    e_pk = jnp.stack([srcp, dstp], axis=1).reshape(
        _NW, 2, _NQ, _QC, _CB).transpose(0, 2, 1, 3, 4)
    xpad = jnp.concatenate(
        [x.astype(jnp.float32), jnp.zeros((_NPAD - _N, _D), jnp.float32)])
    z128 = jnp.zeros((_NPAD, _D), jnp.float32)
    ar = jnp.arange(_HR, dtype=jnp.int32)
    ident = jnp.stack([ar, ar + 128])

    h = _sc_degrees(srcf, dstf, z128, ident)
    xs = _tc_scale(xpad, h)
    p1 = _sc_propagate(xs, e_pk, z128)
    g = _tc_mlp(p1, h, h, W1, W2)
    p2 = _sc_propagate(g, e_pk, z128)
    out = _tc_final(p2, h)
    return out[:_N]
